# baseline (speedup 1.0000x reference)
import functools

import jax
import jax.numpy as jnp
from jax import lax
from jax.experimental import pallas as pl
from jax.experimental.pallas import tpu as pltpu, tpu_sc as plsc

NE = 1_000_000
NR = 1_000
D = 64
B = 16384
R = 2 * B
NC = 2
NS = 16
NW = NC * NS
L = 16
FULLT = NE // 128
TAILBASE = FULLT * 128
TAILN = NE - TAILBASE
TPW = FULLT // NW
TREM = FULLT % NW
CT = 2
MAXG = (TPW + 1 + CT - 1) // CT
ROWSTAGE = 64


def _perm(v, idx):
    return lax.gather(
        v, idx[:, None],
        dimension_numbers=lax.GatherDimensionNumbers(
            offset_dims=(), collapsed_slice_dims=(0,), start_index_map=(0,)),
        slice_sizes=(1,), mode=lax.GatherScatterMode.PROMISE_IN_BOUNDS)


def _sqrt16(v):
    i = lax.bitcast_convert_type(v, jnp.int32)
    i = jnp.int32(0x5F3759DF) - lax.shift_right_logical(i, 1)
    y = lax.bitcast_convert_type(i, jnp.float32)
    for _ in range(3):
        y = y * (1.5 - 0.5 * v * y * y)
    return jnp.where(v > 0.0, v * y, 0.0)


def _phase1_body(hidx, tidx, ent_t, ent_tail, rows_out,
                 creq, reqsorted, hist, starts, cursor,
                 cb0, cb1, rowstage, idxstage, tailbuf,
                 semA, semB, semS):
    w = lax.axis_index("s") * NC + lax.axis_index("c")
    lane = lax.iota(jnp.int32, L)
    zero16 = jnp.zeros((L,), jnp.int32)
    ones16 = zero16 + 1
    lane0 = lane == 0

    tlo = w * TPW + jnp.minimum(w, TREM)
    ntiles = TPW + jnp.where(w < TREM, 1, 0)
    nch = (ntiles + CT - 1) // CT
    thi = tlo + ntiles
    thi_match = jnp.where(w == NW - 1, thi + 1, thi)

    pltpu.sync_copy(hidx, creq.at[pl.ds(0, B)])
    pltpu.sync_copy(tidx, creq.at[pl.ds(B, B)])

    NBIN = 128
    DUMMY = R + 40

    for k in range(NBIN // L):
        hist[pl.ds(k * L, L)] = zero16

    def classify(v):
        e = creq[pl.ds(v * L, L)]
        t = lax.shift_right_logical(e, 7)
        m = (t >= tlo) & (t < thi_match)
        cid = jnp.where(m, lax.shift_right_arithmetic(t - tlo, 1), NBIN - 1)
        return e, m, cid

    def hist_body(v, _):
        _, _, cid = classify(v)
        plsc.addupdate_scatter(hist, [cid], ones16)
        return 0

    lax.fori_loop(0, R // L, hist_body, 0)

    carry = jnp.int32(0)
    for blk in range(NBIN // L):
        h = hist[pl.ds(blk * L, L)]
        x = h
        for sh in (1, 2, 4, 8):
            x = x + jnp.where(lane >= sh,
                              _perm(x, jnp.maximum(lane - sh, 0)), 0)
        excl = x - h + carry
        starts[pl.ds(blk * L, L)] = excl
        cursor[pl.ds(blk * L, L)] = excl
        carry = carry + x[L - 1]

    def scatter_body(v, _):
        e, m, cid = classify(v)
        iloc = e - (tlo + cid * CT) * 128
        pk = iloc * 32768 + (v * L + lane)

        def wcond(st):
            return jnp.any(st)

        def wbody(m_):
            f = plsc.all_reduce_ffs(m_)
            pk_spl = _perm(pk, f)
            cid_spl = _perm(cid, f)
            cur = plsc.load_gather(cursor, [cid_spl])
            tgt = jnp.where(lane0, cur, DUMMY)
            plsc.store_scatter(reqsorted, [tgt], pk_spl)
            plsc.addupdate_scatter(
                cursor, [jnp.where(lane0, cid_spl, NBIN - 1)], ones16)
            return m_ & (lane != f)

        lax.while_loop(wcond, wbody, m)
        return 0

    lax.fori_loop(0, R // L, scatter_body, 0)

    def read_bin(i):
        base = lax.shift_right_logical(i, 4) * L
        vec = starts[pl.ds(base, L)]
        return _perm(vec, zero16 + (i & (L - 1)))[0]

    def fill_sentinels():
        for k in range(ROWSTAGE // L):
            idxstage[0, 0, pl.ds(k * L, L)] = zero16 - 1

    fill_sentinels()

    def fire_scatter():
        pltpu.async_copy(
            rowstage,
            rows_out.at[plsc.Indices(idxstage.at[0, 0], ignored_value=-1)],
            semS,
        ).wait()
        fill_sentinels()

    def walk(buf, rs, re, slot0):
        def vec_body(v, slot):
            pkv = reqsorted[pl.ds(v * L, L)]
            g = v * L + lane
            m = (g >= rs) & (g < re)

            def wcond(st):
                m_, _ = st
                return jnp.any(m_)

            def wbody(st):
                m_, slot_ = st
                f = plsc.all_reduce_ffs(m_)
                pk_spl = _perm(pkv, f)
                pos_spl = pk_spl & 0x7FFF
                iv = lax.shift_right_logical(pk_spl, 15)
                rowslot = slot_ & (ROWSTAGE - 1)
                for q in range(D // L):
                    vq = plsc.load_gather(buf, [lane + q * L, iv])
                    rowstage[rowslot, pl.ds(q * L, L)] = vq
                plsc.store_scatter(idxstage, [zero16, zero16, zero16 + rowslot],
                                   pos_spl)

                @pl.when(rowslot == ROWSTAGE - 1)
                def _():
                    fire_scatter()

                return m_ & (lane != f), slot_ + 1

            _, slot = lax.while_loop(wcond, wbody, (m, slot))
            return slot

        vstart = lax.shift_right_logical(rs, 4)
        vend = lax.shift_right_logical(re + L - 1, 4)
        return lax.fori_loop(vstart, vend, vec_body, slot0)

    def chunk_loop(g, slot):
        for b in range(2):
            c = g * 2 + b
            mybuf = cb0 if b == 0 else cb1
            mysem = semA if b == 0 else semB
            otherbuf = cb1 if b == 0 else cb0
            othersem = semB if b == 0 else semA

            @pl.when(c < nch)
            def _():
                for jt in range(8):
                    pltpu.make_async_copy(
                        ent_t.at[pl.ds(jt * 8, 8), pl.ds(0, CT * 128)],
                        mybuf.at[pl.ds(jt * 8, 8)], mysem).wait()

            @pl.when(c + 1 < nch)
            def _():
                nlo = tlo + (c + 1) * CT
                for jt in range(8):
                    pltpu.async_copy(
                        ent_t.at[pl.ds(jt * 8, 8), pl.ds(nlo * 128, CT * 128)],
                        otherbuf.at[pl.ds(jt * 8, 8)], othersem)

            live = c < nch
            rs = jnp.where(live, read_bin(c), 0)
            re = jnp.where(live, read_bin(c + 1), 0)
            slot = walk(mybuf, rs, re, slot)
        return slot

    for jt in range(8):
        pltpu.async_copy(ent_t.at[pl.ds(jt * 8, 8), pl.ds(tlo * 128, CT * 128)],
                         cb0.at[pl.ds(jt * 8, 8)], semA)
    slot = lax.fori_loop(0, (MAXG + 1) // 2, chunk_loop, jnp.int32(0))

    @pl.when(w == NW - 1)
    def _():
        pltpu.sync_copy(ent_tail, tailbuf)

    is31 = w == NW - 1
    trs = jnp.where(is31, read_bin(nch), 0)
    tre = jnp.where(is31, read_bin(nch + 1), 0)
    slot = walk(tailbuf, trs, tre, slot)

    @pl.when((slot & (ROWSTAGE - 1)) != 0)
    def _():
        fire_scatter()


def _phase2_body(rows, relw, ridx, out, relv, hbuf, tbuf, ridxv, ov, sem):
    w = lax.axis_index("s") * NC + lax.axis_index("c")
    lane = lax.iota(jnp.int32, L)
    base = w * (B // NW)

    pltpu.sync_copy(relw, relv)
    pltpu.sync_copy(ridx.at[pl.ds(base, B // NW)], ridxv)

    perms = [lane ^ sh for sh in (8, 4, 2, 1)]
    masks = [lane == k for k in range(L)]

    def sub(s, _):
        pltpu.sync_copy(rows.at[pl.ds(base + s * 64, 64)], hbuf)
        pltpu.sync_copy(rows.at[pl.ds(B + base + s * 64, 64)], tbuf)
        for v in range(4):
            rv = ridxv[pl.ds(s * 64 + v * L, L)]
            res = jnp.zeros((L,), jnp.float32)
            for l in range(L):
                b = v * L + l
                r_spl = _perm(rv, zero_plus(lane, l))
                acc = None
                for q in range(D // L):
                    relq = plsc.load_gather(relv, [lane + q * L, r_spl])
                    d = hbuf[b, pl.ds(q * L, L)] + relq - tbuf[b, pl.ds(q * L, L)]
                    acc = d * d if acc is None else acc + d * d
                for p in perms:
                    acc = acc + _perm(acc, p)
                res = jnp.where(masks[l], acc, res)
            ov[pl.ds(s * 64 + v * L, L)] = _sqrt16(res)
        return 0

    lax.fori_loop(0, (B // NW) // 64, sub, 0)
    pltpu.sync_copy(ov, out.at[pl.ds(base, B // NW)])


def zero_plus(lane, l):
    return lane * 0 + l


_mesh = plsc.VectorSubcoreMesh(core_axis_name="c", subcore_axis_name="s")
_params = pltpu.CompilerParams(use_tc_tiling_on_sc=True,
                               needs_layout_passes=False)

_phase1 = functools.partial(
    pl.kernel,
    out_type=jax.ShapeDtypeStruct((R, 128), jnp.float32),
    mesh=_mesh,
    compiler_params=_params,
    scratch_types=[
        pltpu.VMEM((R,), jnp.int32),
        pltpu.VMEM((R + 48,), jnp.int32),
        pltpu.VMEM((128,), jnp.int32),
        pltpu.VMEM((144,), jnp.int32),
        pltpu.VMEM((128,), jnp.int32),
        pltpu.VMEM((D, CT * 128), jnp.float32),
        pltpu.VMEM((D, CT * 128), jnp.float32),
        pltpu.VMEM((ROWSTAGE, 128), jnp.float32),
        pltpu.VMEM((1, 1, ROWSTAGE), jnp.int32),
        pltpu.VMEM((D, TAILN), jnp.float32),
        pltpu.SemaphoreType.DMA,
        pltpu.SemaphoreType.DMA,
        pltpu.SemaphoreType.DMA,
    ],
)(_phase1_body)

_phase2 = functools.partial(
    pl.kernel,
    out_type=jax.ShapeDtypeStruct((B,), jnp.float32),
    mesh=_mesh,
    compiler_params=_params,
    scratch_types=[
        pltpu.VMEM((D, NR), jnp.float32),
        pltpu.VMEM((64, 128), jnp.float32),
        pltpu.VMEM((64, 128), jnp.float32),
        pltpu.VMEM((B // NW,), jnp.int32),
        pltpu.VMEM((B // NW,), jnp.float32),
        pltpu.SemaphoreType.DMA,
    ],
)(_phase2_body)


def kernel(head, relation, tail, entity_weight, relation_weight):
    h = head.astype(jnp.int32)
    t = tail.astype(jnp.int32)
    r = relation.astype(jnp.int32)
    ent_t = entity_weight.T
    ent_tail = lax.slice(entity_weight, (TAILBASE, 0), (NE, D)).T
    rows = _phase1(h, t, ent_t, ent_tail)
    return _phase2(rows, relation_weight.T, r)


# speedup vs baseline: 1.0868x; 1.0868x over previous
import functools

import jax
import jax.numpy as jnp
from jax import lax
from jax.experimental import pallas as pl
from jax.experimental.pallas import tpu as pltpu, tpu_sc as plsc

NE = 1_000_000
NR = 1_000
D = 64
B = 16384
R = 2 * B
NC = 2
NS = 16
NW = NC * NS
L = 16
FULLT = NE // 128
TAILBASE = FULLT * 128
TAILN = NE - TAILBASE
TPW = FULLT // NW
TREM = FULLT % NW
CT = 2
MAXG = (TPW + 1 + CT - 1) // CT
ROWSTAGE = 64


def _perm(v, idx):
    return lax.gather(
        v, idx[:, None],
        dimension_numbers=lax.GatherDimensionNumbers(
            offset_dims=(), collapsed_slice_dims=(0,), start_index_map=(0,)),
        slice_sizes=(1,), mode=lax.GatherScatterMode.PROMISE_IN_BOUNDS)


def _sqrt16(v):
    i = lax.bitcast_convert_type(v, jnp.int32)
    i = jnp.int32(0x5F3759DF) - lax.shift_right_logical(i, 1)
    y = lax.bitcast_convert_type(i, jnp.float32)
    for _ in range(3):
        y = y * (1.5 - 0.5 * v * y * y)
    return jnp.where(v > 0.0, v * y, 0.0)


def _phase1_body(hidx, tidx, ent_t, ent_tail, rows_out,
                 creq, reqsorted, hist, starts, cursor,
                 cb0, cb1, rowstage, idxstage, tailbuf,
                 semA, semB, semS):
    w = lax.axis_index("s") * NC + lax.axis_index("c")
    lane = lax.iota(jnp.int32, L)
    zero16 = jnp.zeros((L,), jnp.int32)
    ones16 = zero16 + 1
    lane0 = lane == 0

    tlo = w * TPW + jnp.minimum(w, TREM)
    ntiles = TPW + jnp.where(w < TREM, 1, 0)
    nch = (ntiles + CT - 1) // CT
    thi = tlo + ntiles
    thi_match = jnp.where(w == NW - 1, thi + 1, thi)

    pltpu.sync_copy(hidx, creq.at[pl.ds(0, B)])
    pltpu.sync_copy(tidx, creq.at[pl.ds(B, B)])

    NBIN = 128
    DUMMY = R + 40

    for k in range(NBIN // L):
        hist[pl.ds(k * L, L)] = zero16

    def classify(v):
        e = creq[pl.ds(v * L, L)]
        t = lax.shift_right_logical(e, 7)
        m = (t >= tlo) & (t < thi_match)
        cid = jnp.where(m, lax.shift_right_arithmetic(t - tlo, 1), NBIN - 1)
        return e, m, cid

    def hist_body(v, _):
        _, _, cid = classify(v)
        plsc.addupdate_scatter(hist, [cid], ones16)
        return 0

    lax.fori_loop(0, R // L, hist_body, 0)

    carry = jnp.int32(0)
    for blk in range(NBIN // L):
        h = hist[pl.ds(blk * L, L)]
        x = h
        for sh in (1, 2, 4, 8):
            x = x + jnp.where(lane >= sh,
                              _perm(x, jnp.maximum(lane - sh, 0)), 0)
        excl = x - h + carry
        starts[pl.ds(blk * L, L)] = excl
        cursor[pl.ds(blk * L, L)] = excl
        carry = carry + x[L - 1]

    def scatter_body(v, _):
        e, m, cid = classify(v)
        iloc = e - (tlo + cid * CT) * 128
        pk = iloc * 32768 + (v * L + lane)

        def wcond(st):
            return jnp.any(st)

        def wbody(m_):
            f = plsc.all_reduce_ffs(m_)
            pk_spl = _perm(pk, f)
            cid_spl = _perm(cid, f)
            cur = plsc.load_gather(cursor, [cid_spl])
            tgt = jnp.where(lane0, cur, DUMMY)
            plsc.store_scatter(reqsorted, [tgt], pk_spl)
            plsc.addupdate_scatter(
                cursor, [jnp.where(lane0, cid_spl, NBIN - 1)], ones16)
            return m_ & (lane != f)

        lax.while_loop(wcond, wbody, m)
        return 0

    lax.fori_loop(0, R // L, scatter_body, 0)

    def read_bin(i):
        base = lax.shift_right_logical(i, 4) * L
        vec = starts[pl.ds(base, L)]
        return _perm(vec, zero16 + (i & (L - 1)))[0]

    def fill_sentinels():
        for k in range(ROWSTAGE // L):
            idxstage[0, 0, pl.ds(k * L, L)] = zero16 - 1

    fill_sentinels()

    def fire_scatter():
        pltpu.async_copy(
            rowstage,
            rows_out.at[plsc.Indices(idxstage.at[0, 0], ignored_value=-1)],
            semS,
        ).wait()
        fill_sentinels()

    def walk(buf, rs, re, slot0):
        def vec_body(v, slot):
            pkv = reqsorted[pl.ds(v * L, L)]
            g = v * L + lane
            m = (g >= rs) & (g < re)

            def wcond(st):
                m_, _ = st
                return jnp.any(m_)

            def wbody(st):
                m_, slot_ = st
                f = plsc.all_reduce_ffs(m_)
                pk_spl = _perm(pkv, f)
                pos_spl = pk_spl & 0x7FFF
                iv = lax.shift_right_logical(pk_spl, 15)
                rowslot = slot_ & (ROWSTAGE - 1)
                for q in range(D // L):
                    vq = plsc.load_gather(buf, [lane + q * L, iv])
                    rowstage[rowslot, pl.ds(q * L, L)] = vq
                plsc.store_scatter(idxstage, [zero16, zero16, zero16 + rowslot],
                                   pos_spl)

                @pl.when(rowslot == ROWSTAGE - 1)
                def _():
                    fire_scatter()

                return m_ & (lane != f), slot_ + 1

            _, slot = lax.while_loop(wcond, wbody, (m, slot))
            return slot

        vstart = lax.shift_right_logical(rs, 4)
        vend = lax.shift_right_logical(re + L - 1, 4)
        return lax.fori_loop(vstart, vend, vec_body, slot0)

    def chunk_loop(g, slot):
        for b in range(2):
            c = g * 2 + b
            mybuf = cb0 if b == 0 else cb1
            mysem = semA if b == 0 else semB
            otherbuf = cb1 if b == 0 else cb0
            othersem = semB if b == 0 else semA

            @pl.when(c + 1 < nch)
            def _():
                nlo = tlo + (c + 1) * CT
                for jt in range(8):
                    pltpu.async_copy(
                        ent_t.at[pl.ds(jt * 8, 8), pl.ds(nlo * 128, CT * 128)],
                        otherbuf.at[pl.ds(jt * 8, 8)], othersem)

            @pl.when(c < nch)
            def _():
                for jt in range(8):
                    pltpu.make_async_copy(
                        ent_t.at[pl.ds(jt * 8, 8), pl.ds(0, CT * 128)],
                        mybuf.at[pl.ds(jt * 8, 8)], mysem).wait()

            live = c < nch
            rs = jnp.where(live, read_bin(c), 0)
            re = jnp.where(live, read_bin(c + 1), 0)
            slot = walk(mybuf, rs, re, slot)
        return slot

    for jt in range(8):
        pltpu.async_copy(ent_t.at[pl.ds(jt * 8, 8), pl.ds(tlo * 128, CT * 128)],
                         cb0.at[pl.ds(jt * 8, 8)], semA)
    slot = lax.fori_loop(0, (MAXG + 1) // 2, chunk_loop, jnp.int32(0))

    @pl.when(w == NW - 1)
    def _():
        pltpu.sync_copy(ent_tail, tailbuf)

    is31 = w == NW - 1
    trs = jnp.where(is31, read_bin(nch), 0)
    tre = jnp.where(is31, read_bin(nch + 1), 0)
    slot = walk(tailbuf, trs, tre, slot)

    @pl.when((slot & (ROWSTAGE - 1)) != 0)
    def _():
        fire_scatter()


def _phase2_body(rows, relw, ridx, out, relv, hbuf, tbuf, ridxv, ov, sem):
    w = lax.axis_index("s") * NC + lax.axis_index("c")
    lane = lax.iota(jnp.int32, L)
    base = w * (B // NW)

    pltpu.sync_copy(relw, relv)
    pltpu.sync_copy(ridx.at[pl.ds(base, B // NW)], ridxv)

    perms = [lane ^ sh for sh in (8, 4, 2, 1)]
    masks = [lane == k for k in range(L)]

    def sub(s, _):
        pltpu.sync_copy(rows.at[pl.ds(base + s * 64, 64)], hbuf)
        pltpu.sync_copy(rows.at[pl.ds(B + base + s * 64, 64)], tbuf)
        for v in range(4):
            rv = ridxv[pl.ds(s * 64 + v * L, L)]
            res = jnp.zeros((L,), jnp.float32)
            for l in range(L):
                b = v * L + l
                r_spl = _perm(rv, zero_plus(lane, l))
                acc = None
                for q in range(D // L):
                    relq = plsc.load_gather(relv, [lane + q * L, r_spl])
                    d = hbuf[b, pl.ds(q * L, L)] + relq - tbuf[b, pl.ds(q * L, L)]
                    acc = d * d if acc is None else acc + d * d
                for p in perms:
                    acc = acc + _perm(acc, p)
                res = jnp.where(masks[l], acc, res)
            ov[pl.ds(s * 64 + v * L, L)] = _sqrt16(res)
        return 0

    lax.fori_loop(0, (B // NW) // 64, sub, 0)
    pltpu.sync_copy(ov, out.at[pl.ds(base, B // NW)])


def zero_plus(lane, l):
    return lane * 0 + l


_mesh = plsc.VectorSubcoreMesh(core_axis_name="c", subcore_axis_name="s")
_params = pltpu.CompilerParams(use_tc_tiling_on_sc=True,
                               needs_layout_passes=False)

_phase1 = functools.partial(
    pl.kernel,
    out_type=jax.ShapeDtypeStruct((R, 128), jnp.float32),
    mesh=_mesh,
    compiler_params=_params,
    scratch_types=[
        pltpu.VMEM((R,), jnp.int32),
        pltpu.VMEM((R + 48,), jnp.int32),
        pltpu.VMEM((128,), jnp.int32),
        pltpu.VMEM((144,), jnp.int32),
        pltpu.VMEM((128,), jnp.int32),
        pltpu.VMEM((D, CT * 128), jnp.float32),
        pltpu.VMEM((D, CT * 128), jnp.float32),
        pltpu.VMEM((ROWSTAGE, 128), jnp.float32),
        pltpu.VMEM((1, 1, ROWSTAGE), jnp.int32),
        pltpu.VMEM((D, TAILN), jnp.float32),
        pltpu.SemaphoreType.DMA,
        pltpu.SemaphoreType.DMA,
        pltpu.SemaphoreType.DMA,
    ],
)(_phase1_body)

_phase2 = functools.partial(
    pl.kernel,
    out_type=jax.ShapeDtypeStruct((B,), jnp.float32),
    mesh=_mesh,
    compiler_params=_params,
    scratch_types=[
        pltpu.VMEM((D, NR), jnp.float32),
        pltpu.VMEM((64, 128), jnp.float32),
        pltpu.VMEM((64, 128), jnp.float32),
        pltpu.VMEM((B // NW,), jnp.int32),
        pltpu.VMEM((B // NW,), jnp.float32),
        pltpu.SemaphoreType.DMA,
    ],
)(_phase2_body)


def kernel(head, relation, tail, entity_weight, relation_weight):
    h = head.astype(jnp.int32)
    t = tail.astype(jnp.int32)
    r = relation.astype(jnp.int32)
    ent_t = entity_weight.T
    ent_tail = lax.slice(entity_weight, (TAILBASE, 0), (NE, D)).T
    rows = _phase1(h, t, ent_t, ent_tail)
    return _phase2(rows, relation_weight.T, r)


# speedup vs baseline: 1.1921x; 1.0969x over previous
import functools

import jax
import jax.numpy as jnp
from jax import lax
from jax.experimental import pallas as pl
from jax.experimental.pallas import tpu as pltpu, tpu_sc as plsc

NE = 1_000_000
NR = 1_000
D = 64
B = 16384
R = 2 * B
NC = 2
NS = 16
NW = NC * NS
L = 16
FULLT = NE // 128
TAILBASE = FULLT * 128
TAILN = NE - TAILBASE
TPW = FULLT // NW
TREM = FULLT % NW
CT = 2
MAXG = (TPW + 1 + CT - 1) // CT
ROWSTAGE = 64


def _perm(v, idx):
    return lax.gather(
        v, idx[:, None],
        dimension_numbers=lax.GatherDimensionNumbers(
            offset_dims=(), collapsed_slice_dims=(0,), start_index_map=(0,)),
        slice_sizes=(1,), mode=lax.GatherScatterMode.PROMISE_IN_BOUNDS)


def _sqrt16(v):
    i = lax.bitcast_convert_type(v, jnp.int32)
    i = jnp.int32(0x5F3759DF) - lax.shift_right_logical(i, 1)
    y = lax.bitcast_convert_type(i, jnp.float32)
    for _ in range(3):
        y = y * (1.5 - 0.5 * v * y * y)
    return jnp.where(v > 0.0, v * y, 0.0)


def _phase1_body(hidx, tidx, ent_t, ent_tail, rows_out,
                 creq, reqsorted, hist, starts, cursor,
                 cb0, cb1, cb2, rowstage, idxstage,
                 semA, semB, semC, semS):
    w = lax.axis_index("s") * NC + lax.axis_index("c")
    lane = lax.iota(jnp.int32, L)
    zero16 = jnp.zeros((L,), jnp.int32)
    ones16 = zero16 + 1
    lane0 = lane == 0

    tlo = w * TPW + jnp.minimum(w, TREM)
    ntiles = TPW + jnp.where(w < TREM, 1, 0)
    nch = (ntiles + CT - 1) // CT
    thi = tlo + ntiles
    thi_match = jnp.where(w == NW - 1, thi + 1, thi)

    pltpu.sync_copy(hidx, creq.at[pl.ds(0, B)])
    pltpu.sync_copy(tidx, creq.at[pl.ds(B, B)])

    NBIN = 128
    DUMMY = R + 40

    for k in range(NBIN // L):
        hist[pl.ds(k * L, L)] = zero16

    def classify(v):
        e = creq[pl.ds(v * L, L)]
        t = lax.shift_right_logical(e, 7)
        m = (t >= tlo) & (t < thi_match)
        cid = jnp.where(m, lax.shift_right_arithmetic(t - tlo, 1), NBIN - 1)
        return e, m, cid

    def hist_body(v, _):
        _, _, cid = classify(v)
        plsc.addupdate_scatter(hist, [cid], ones16)
        return 0

    lax.fori_loop(0, R // L, hist_body, 0)

    carry = jnp.int32(0)
    for blk in range(NBIN // L):
        h = hist[pl.ds(blk * L, L)]
        x = h
        for sh in (1, 2, 4, 8):
            x = x + jnp.where(lane >= sh,
                              _perm(x, jnp.maximum(lane - sh, 0)), 0)
        excl = x - h + carry
        starts[pl.ds(blk * L, L)] = excl
        cursor[pl.ds(blk * L, L)] = excl
        carry = carry + x[L - 1]

    def scatter_body(v, _):
        e, m, cid = classify(v)
        iloc = e - (tlo + cid * CT) * 128
        pk = iloc * 32768 + (v * L + lane)

        def wcond(st):
            return jnp.any(st)

        def wbody(m_):
            f = plsc.all_reduce_ffs(m_)
            pk_spl = _perm(pk, f)
            cid_spl = _perm(cid, f)
            cur = plsc.load_gather(cursor, [cid_spl])
            tgt = jnp.where(lane0, cur, DUMMY)
            plsc.store_scatter(reqsorted, [tgt], pk_spl)
            plsc.addupdate_scatter(
                cursor, [jnp.where(lane0, cid_spl, NBIN - 1)], ones16)
            return m_ & (lane != f)

        lax.while_loop(wcond, wbody, m)
        return 0

    lax.fori_loop(0, R // L, scatter_body, 0)

    def read_bin(i):
        base = lax.shift_right_logical(i, 4) * L
        vec = starts[pl.ds(base, L)]
        return _perm(vec, zero16 + (i & (L - 1)))[0]

    def fill_sentinels():
        for k in range(ROWSTAGE // L):
            idxstage[0, 0, pl.ds(k * L, L)] = zero16 - 1

    fill_sentinels()

    def fire_scatter():
        pltpu.async_copy(
            rowstage,
            rows_out.at[plsc.Indices(idxstage.at[0, 0], ignored_value=-1)],
            semS,
        ).wait()
        fill_sentinels()

    def walk(buf, rs, re, slot0):
        def vec_body(v, slot):
            pkv = reqsorted[pl.ds(v * L, L)]
            g = v * L + lane
            m = (g >= rs) & (g < re)

            def wcond(st):
                m_, _ = st
                return jnp.any(m_)

            def wbody(st):
                m_, slot_ = st
                f = plsc.all_reduce_ffs(m_)
                pk_spl = _perm(pkv, f)
                pos_spl = pk_spl & 0x7FFF
                iv = lax.shift_right_logical(pk_spl, 15)
                rowslot = slot_ & (ROWSTAGE - 1)
                for q in range(D // L):
                    vq = plsc.load_gather(buf, [lane + q * L, iv])
                    rowstage[rowslot, pl.ds(q * L, L)] = vq
                plsc.store_scatter(idxstage, [zero16, zero16, zero16 + rowslot],
                                   pos_spl)

                @pl.when(rowslot == ROWSTAGE - 1)
                def _():
                    fire_scatter()

                return m_ & (lane != f), slot_ + 1

            _, slot = lax.while_loop(wcond, wbody, (m, slot))
            return slot

        vstart = lax.shift_right_logical(rs, 4)
        vend = lax.shift_right_logical(re + L - 1, 4)
        return lax.fori_loop(vstart, vend, vec_body, slot0)

    def chunk_loop(g, slot):
        for b in range(3):
            c = g * 3 + b
            bufs = (cb0, cb1, cb2)
            sems = (semA, semB, semC)
            mybuf, mysem = bufs[b], sems[b]
            nxt = (b + 2) % 3

            @pl.when(c + 2 < nch)
            def _():
                nlo = tlo + (c + 2) * CT
                for jt in range(8):
                    pltpu.async_copy(
                        ent_t.at[pl.ds(jt * 8, 8), pl.ds(nlo * 128, CT * 128)],
                        bufs[nxt].at[pl.ds(jt * 8, 8)], sems[nxt])

            @pl.when(c < nch)
            def _():
                for jt in range(8):
                    pltpu.make_async_copy(
                        ent_t.at[pl.ds(jt * 8, 8), pl.ds(0, CT * 128)],
                        mybuf.at[pl.ds(jt * 8, 8)], mysem).wait()

            live = c < nch
            rs = jnp.where(live, read_bin(c), 0)
            re = jnp.where(live, read_bin(c + 1), 0)
            slot = walk(mybuf, rs, re, slot)
        return slot

    for cc, (bb, ss) in ((0, (cb0, semA)), (1, (cb1, semB))):
        @pl.when(cc < nch)
        def _():
            for jt in range(8):
                pltpu.async_copy(
                    ent_t.at[pl.ds(jt * 8, 8),
                             pl.ds((tlo + cc) * 128, CT * 128)],
                    bb.at[pl.ds(jt * 8, 8)], ss)
    slot = lax.fori_loop(0, (MAXG + 2) // 3, chunk_loop, jnp.int32(0))

    @pl.when(w == NW - 1)
    def _():
        pltpu.sync_copy(ent_tail, cb0.at[:, pl.ds(0, 128)])

    is31 = w == NW - 1
    trs = jnp.where(is31, read_bin(nch), 0)
    tre = jnp.where(is31, read_bin(nch + 1), 0)
    slot = walk(cb0, trs, tre, slot)

    @pl.when((slot & (ROWSTAGE - 1)) != 0)
    def _():
        fire_scatter()


def _phase2_body(rows, relw, ridx, out, relv, hbuf, tbuf, ridxv, ov, sem):
    w = lax.axis_index("s") * NC + lax.axis_index("c")
    lane = lax.iota(jnp.int32, L)
    base = w * (B // NW)

    pltpu.sync_copy(relw, relv)
    pltpu.sync_copy(ridx.at[pl.ds(base, B // NW)], ridxv)

    perms = [lane ^ sh for sh in (8, 4, 2, 1)]
    masks = [lane == k for k in range(L)]

    def sub(s, _):
        pltpu.sync_copy(rows.at[pl.ds(base + s * 64, 64)], hbuf)
        pltpu.sync_copy(rows.at[pl.ds(B + base + s * 64, 64)], tbuf)
        for v in range(4):
            rv = ridxv[pl.ds(s * 64 + v * L, L)]
            res = jnp.zeros((L,), jnp.float32)
            for l in range(L):
                b = v * L + l
                r_spl = _perm(rv, zero_plus(lane, l))
                acc = None
                for q in range(D // L):
                    relq = plsc.load_gather(relv, [lane + q * L, r_spl])
                    d = hbuf[b, pl.ds(q * L, L)] + relq - tbuf[b, pl.ds(q * L, L)]
                    acc = d * d if acc is None else acc + d * d
                for p in perms:
                    acc = acc + _perm(acc, p)
                res = jnp.where(masks[l], acc, res)
            ov[pl.ds(s * 64 + v * L, L)] = _sqrt16(res)
        return 0

    lax.fori_loop(0, (B // NW) // 64, sub, 0)
    pltpu.sync_copy(ov, out.at[pl.ds(base, B // NW)])


def zero_plus(lane, l):
    return lane * 0 + l


_mesh = plsc.VectorSubcoreMesh(core_axis_name="c", subcore_axis_name="s")
_params = pltpu.CompilerParams(use_tc_tiling_on_sc=True,
                               needs_layout_passes=False)

_phase1 = functools.partial(
    pl.kernel,
    out_type=jax.ShapeDtypeStruct((R, 128), jnp.float32),
    mesh=_mesh,
    compiler_params=_params,
    scratch_types=[
        pltpu.VMEM((R,), jnp.int32),
        pltpu.VMEM((R + 48,), jnp.int32),
        pltpu.VMEM((128,), jnp.int32),
        pltpu.VMEM((144,), jnp.int32),
        pltpu.VMEM((128,), jnp.int32),
        pltpu.VMEM((D, CT * 128), jnp.float32),
        pltpu.VMEM((D, CT * 128), jnp.float32),
        pltpu.VMEM((D, CT * 128), jnp.float32),
        pltpu.VMEM((ROWSTAGE, 128), jnp.float32),
        pltpu.VMEM((1, 1, ROWSTAGE), jnp.int32),
        pltpu.SemaphoreType.DMA,
        pltpu.SemaphoreType.DMA,
        pltpu.SemaphoreType.DMA,
        pltpu.SemaphoreType.DMA,
    ],
)(_phase1_body)

_phase2 = functools.partial(
    pl.kernel,
    out_type=jax.ShapeDtypeStruct((B,), jnp.float32),
    mesh=_mesh,
    compiler_params=_params,
    scratch_types=[
        pltpu.VMEM((D, NR), jnp.float32),
        pltpu.VMEM((64, 128), jnp.float32),
        pltpu.VMEM((64, 128), jnp.float32),
        pltpu.VMEM((B // NW,), jnp.int32),
        pltpu.VMEM((B // NW,), jnp.float32),
        pltpu.SemaphoreType.DMA,
    ],
)(_phase2_body)


def kernel(head, relation, tail, entity_weight, relation_weight):
    h = head.astype(jnp.int32)
    t = tail.astype(jnp.int32)
    r = relation.astype(jnp.int32)
    ent_t = entity_weight.T
    ent_tail = jnp.pad(lax.slice(entity_weight, (TAILBASE, 0), (NE, D)).T,
                       ((0, 0), (0, 128 - TAILN)))
    rows = _phase1(h, t, ent_t, ent_tail)
    return _phase2(rows, relation_weight.T, r)
